# baseline (speedup 1.0000x reference)
import functools

import jax
import jax.numpy as jnp
from jax import lax
from jax.experimental import pallas as pl
from jax.experimental.pallas import tpu as pltpu
from jax.experimental.pallas import tpu_sc as plsc

NUM_USER = 100000
NUM_ITEM = 100000
EMBED = 64
B = 4096
WEIGHT_DECAY = 0.0001

NC = 2
NS = 16
NW = NC * NS
BPW = B // NW


def _sc_gather3_body(user_h, pos_h, neg_h, utab_h, itab_h,
                     ue_h, pe_h, ne_h,
                     idx_u, idx_p, idx_n, rows_u, rows_p, rows_n, sem):
    wid = lax.axis_index("s") * NC + lax.axis_index("c")
    base = wid * BPW
    pltpu.sync_copy(user_h.at[pl.ds(base, BPW)], idx_u)
    pltpu.sync_copy(pos_h.at[pl.ds(base, BPW)], idx_p)
    pltpu.sync_copy(neg_h.at[pl.ds(base, BPW)], idx_n)
    cu = pltpu.async_copy(utab_h.at[idx_u], rows_u, sem)
    cp = pltpu.async_copy(itab_h.at[idx_p], rows_p, sem)
    cn = pltpu.async_copy(itab_h.at[idx_n], rows_n, sem)
    cu.wait()
    pltpu.sync_copy(rows_u, ue_h.at[pl.ds(base, BPW)])
    cp.wait()
    pltpu.sync_copy(rows_p, pe_h.at[pl.ds(base, BPW)])
    cn.wait()
    pltpu.sync_copy(rows_n, ne_h.at[pl.ds(base, BPW)])


@functools.cache
def _sc_gather3():
    return pl.kernel(
        _sc_gather3_body,
        out_type=[jax.ShapeDtypeStruct((B, EMBED), jnp.float32)] * 3,
        mesh=plsc.VectorSubcoreMesh(core_axis_name="c", subcore_axis_name="s",
                                    num_cores=NC, num_subcores=NS),
        scratch_types=[
            pltpu.VMEM((BPW,), jnp.int32),
            pltpu.VMEM((BPW,), jnp.int32),
            pltpu.VMEM((BPW,), jnp.int32),
            pltpu.VMEM((BPW, EMBED), jnp.float32),
            pltpu.VMEM((BPW, EMBED), jnp.float32),
            pltpu.VMEM((BPW, EMBED), jnp.float32),
            pltpu.SemaphoreType.DMA,
        ],
        compiler_params=pltpu.CompilerParams(use_tc_tiling_on_sc=False),
    )


def _tc_scalars_body(ue_ref, pe_ref, ne_ref, loss_ref, nlp_ref, reg_ref):
    ue = ue_ref[...]
    pe = pe_ref[...]
    ne = ne_ref[...]
    pos_out = jnp.sum(ue * pe, axis=1, keepdims=True)
    neg_out = jnp.sum(ue * ne, axis=1, keepdims=True)
    out = pos_out - neg_out
    log_prob = jnp.sum(jax.nn.log_sigmoid(out))
    reg = WEIGHT_DECAY * (jnp.sum(ue * ue) + jnp.sum(pe * pe)
                          + jnp.sum(ne * ne))
    nlp_ref[0, 0] = -log_prob
    reg_ref[0, 0] = reg
    loss_ref[0, 0] = -log_prob + reg


def _tc_scalars(ue, pe, ne):
    return pl.pallas_call(
        _tc_scalars_body,
        out_shape=[jax.ShapeDtypeStruct((1, 1), jnp.float32)] * 3,
        out_specs=[pl.BlockSpec(memory_space=pltpu.SMEM)] * 3,
    )(ue, pe, ne)


def kernel(user, pos, neg, history, history_mask, user_table, item_table):
    ue, pe, ne = _sc_gather3()(user, pos, neg, user_table, item_table)
    loss, nlp, reg = _tc_scalars(ue, pe, ne)
    return (loss[0, 0], nlp[0, 0], reg[0, 0], ue, pe, ne)
